# baseline (speedup 1.0000x reference)
import functools

import jax
import jax.numpy as jnp
from jax import lax
from jax.experimental import pallas as pl
from jax.experimental.pallas import tpu as pltpu
from jax.experimental.pallas import tpu_sc as plsc

VOCAB = 1000000
D = 64
BATCH = 16384
EPS = 1e-10

NW = 32
ROWS_W = BATCH // NW
C = 128
NCHUNK = ROWS_W // C
NIDX = 5 * C
GSUB = 128
NG = NIDX // GSUB
LANES = 16


def _rsqrt_nr(x):
    i = lax.bitcast_convert_type(x, jnp.int32)
    i = jnp.int32(0x5F3759DF) - lax.shift_right_logical(i, 1)
    y = lax.bitcast_convert_type(i, jnp.float32)
    for _ in range(3):
        y = y * (1.5 - 0.5 * x * y * y)
    return y


def _make_sc_kernel():
    mesh = plsc.VectorSubcoreMesh(core_axis_name="c", subcore_axis_name="s")

    @functools.partial(
        pl.kernel,
        mesh=mesh,
        compiler_params=pltpu.CompilerParams(
            needs_layout_passes=False, use_tc_tiling_on_sc=False),
        out_type=jax.ShapeDtypeStruct((BATCH * 3,), jnp.float32),
        scratch_types=[
            pltpu.VMEM((2, NIDX), jnp.int32),
            pltpu.VMEM((2, NIDX, D), jnp.float32),
            pltpu.VMEM((3 * C,), jnp.float32),
            pltpu.SemaphoreType.DMA,
            pltpu.SemaphoreType.DMA,
        ],
    )
    def sc_fn(idx_hbm, table_hbm, out_hbm, idx_v, gbuf, outbuf,
              sem0, sem1):
        wid = lax.axis_index("s") * 2 + lax.axis_index("c")
        base_row = wid * ROWS_W
        sems = (sem0, sem1)

        def start_gather(t, slot):
            off = (base_row + t * C) * 5
            pltpu.sync_copy(idx_hbm.at[pl.ds(off, NIDX)], idx_v.at[slot])
            handles = []
            for k in range(NG):
                handles.append(pltpu.async_copy(
                    table_hbm.at[idx_v.at[slot, pl.ds(k * GSUB, GSUB)]],
                    gbuf.at[slot, pl.ds(k * GSUB, GSUB)],
                    sems[slot]))
            return handles

        def compute_chunk(slot, t):
            lane = lax.iota(jnp.int32, LANES)
            slotv = jnp.full((LANES,), slot, jnp.int32)
            zeros = jnp.zeros((LANES,), jnp.float32)

            def group(g, carry):
                row0 = 5 * (g * LANES + lane)
                rx, ry, r1v, r2v, r3v = (row0 + op for op in range(5))

                def dbody(d, acc):
                    hh, d1, d2, d3, s1, s2, s3 = acc
                    dv = jnp.full((LANES,), d, jnp.int32)
                    x = plsc.load_gather(gbuf, [slotv, rx, dv])
                    y = plsc.load_gather(gbuf, [slotv, ry, dv])
                    h = x - y
                    r1 = plsc.load_gather(gbuf, [slotv, r1v, dv])
                    r2 = plsc.load_gather(gbuf, [slotv, r2v, dv])
                    r3 = plsc.load_gather(gbuf, [slotv, r3v, dv])
                    return (hh + h * h, d1 + h * r1, d2 + h * r2,
                            d3 + h * r3, s1 + r1 * r1, s2 + r2 * r2,
                            s3 + r3 * r3)

                hh, d1, d2, d3, s1, s2, s3 = lax.fori_loop(
                    0, D, dbody, (zeros,) * 7)
                nh = jnp.maximum(hh * _rsqrt_nr(hh), EPS)
                ob = (g * LANES + lane) * 3
                for i, (dd, ss) in enumerate(
                        ((d1, s1), (d2, s2), (d3, s3))):
                    nr = jnp.maximum(ss * _rsqrt_nr(ss), EPS)
                    plsc.store_scatter(outbuf, [ob + i], dd / (nh * nr))
                return carry

            lax.fori_loop(0, C // LANES, group, 0)
            pltpu.sync_copy(
                outbuf, out_hbm.at[pl.ds((base_row + t * C) * 3, 3 * C)])

        handles = [None, None]
        handles[0] = start_gather(0, 0)
        for t in range(NCHUNK):
            slot = t % 2
            if t + 1 < NCHUNK:
                handles[1 - slot] = start_gather(t + 1, 1 - slot)
            for h in handles[slot]:
                h.wait()
            compute_chunk(slot, t)

    return sc_fn


_SC_KERNEL = _make_sc_kernel()


def kernel(input, onepole, four, table):
    idx_flat = input.astype(jnp.int32).reshape(-1)
    out_flat = _SC_KERNEL(idx_flat, table)
    return out_flat.reshape(BATCH, 3)


# speedup vs baseline: 1.0911x; 1.0911x over previous
import functools

import jax
import jax.numpy as jnp
from jax import lax
from jax.experimental import pallas as pl
from jax.experimental.pallas import tpu as pltpu
from jax.experimental.pallas import tpu_sc as plsc

VOCAB = 1000000
D = 64
BATCH = 16384
EPS = 1e-10

NW = 32
ROWS_W = BATCH // NW
C = 128
NCHUNK = ROWS_W // C
NIDX = 5 * C
GSUB = 128
NG = NIDX // GSUB
LANES = 16


def _rsqrt_nr(x):
    i = lax.bitcast_convert_type(x, jnp.int32)
    i = jnp.int32(0x5F3759DF) - lax.shift_right_logical(i, 1)
    y = lax.bitcast_convert_type(i, jnp.float32)
    for _ in range(3):
        y = y * (1.5 - 0.5 * x * y * y)
    return y


def _make_sc_kernel():
    mesh = plsc.VectorSubcoreMesh(core_axis_name="c", subcore_axis_name="s")

    @functools.partial(
        pl.kernel,
        mesh=mesh,
        compiler_params=pltpu.CompilerParams(
            needs_layout_passes=False, use_tc_tiling_on_sc=False),
        out_type=jax.ShapeDtypeStruct((BATCH * 3,), jnp.float32),
        scratch_types=[
            pltpu.VMEM((2, NIDX), jnp.int32),
            pltpu.VMEM((2, NIDX, D), jnp.float32),
            pltpu.VMEM((7 * C,), jnp.float32),
            pltpu.VMEM((3 * C,), jnp.float32),
            pltpu.SemaphoreType.DMA,
            pltpu.SemaphoreType.DMA,
        ],
    )
    def sc_fn(idx_hbm, table_hbm, out_hbm, idx_v, gbuf, stats, outbuf,
              sem0, sem1):
        wid = lax.axis_index("s") * 2 + lax.axis_index("c")
        base_row = wid * ROWS_W
        sems = (sem0, sem1)

        def start_gather(t, slot):
            off = (base_row + t * C) * 5
            pltpu.sync_copy(idx_hbm.at[pl.ds(off, NIDX)], idx_v.at[slot])
            handles = []
            for k in range(NG):
                handles.append(pltpu.async_copy(
                    table_hbm.at[idx_v.at[slot, pl.ds(k * GSUB, GSUB)]],
                    gbuf.at[slot, pl.ds(k * GSUB, GSUB)],
                    sems[slot]))
            return handles

        def compute_chunk(slot, t):
            lane = lax.iota(jnp.int32, LANES)
            m15 = lane == (LANES - 1)

            def row(c, carry):
                r0 = 5 * c
                cb = jnp.full((LANES,), c, jnp.int32)

                def put(k, vec):
                    tot = jnp.cumsum(vec)
                    plsc.store_scatter(stats, [cb + (k * C)], tot, mask=m15)

                hv = []
                hh = None
                for j in range(D // LANES):
                    x = gbuf[slot, r0, pl.ds(LANES * j, LANES)]
                    y = gbuf[slot, r0 + 1, pl.ds(LANES * j, LANES)]
                    h = x - y
                    hv.append(h)
                    hh = h * h if hh is None else hh + h * h
                put(0, hh)
                for i in range(3):
                    dv = None
                    rv = None
                    for j in range(D // LANES):
                        r = gbuf[slot, r0 + 2 + i, pl.ds(LANES * j, LANES)]
                        dv = hv[j] * r if dv is None else dv + hv[j] * r
                        rv = r * r if rv is None else rv + r * r
                    put(1 + i, dv)
                    put(4 + i, rv)
                return carry

            lax.fori_loop(0, C, row, 0)

            for g in range(C // LANES):
                hh = stats[pl.ds(LANES * g, LANES)]
                nh = jnp.maximum(hh * _rsqrt_nr(hh), EPS)
                ob = (g * LANES + lane) * 3
                for i in range(3):
                    dd = stats[pl.ds((1 + i) * C + LANES * g, LANES)]
                    rr = stats[pl.ds((4 + i) * C + LANES * g, LANES)]
                    nr = jnp.maximum(rr * _rsqrt_nr(rr), EPS)
                    plsc.store_scatter(outbuf, [ob + i], dd / (nh * nr))
            pltpu.sync_copy(
                outbuf, out_hbm.at[pl.ds((base_row + t * C) * 3, 3 * C)])

        handles = [None, None]
        handles[0] = start_gather(0, 0)
        for t in range(NCHUNK):
            slot = t % 2
            if t + 1 < NCHUNK:
                handles[1 - slot] = start_gather(t + 1, 1 - slot)
            for h in handles[slot]:
                h.wait()
            compute_chunk(slot, t)

    return sc_fn


_SC_KERNEL = _make_sc_kernel()


def kernel(input, onepole, four, table):
    idx_flat = input.astype(jnp.int32).reshape(-1)
    out_flat = _SC_KERNEL(idx_flat, table)
    return out_flat.reshape(BATCH, 3)
